# baseline (speedup 1.0000x reference)
import jax
import jax.numpy as jnp
from jax import lax
from jax.experimental import pallas as pl
from jax.experimental.pallas import tpu as pltpu
from jax.experimental.pallas import tpu_sc as plsc

EPS = 1e-6
NC = 2
NS = 16
NW = NC * NS
G = 16


def _sel_kernel(act_ref, inp_ref, out_ref):
    b = pl.program_id(0)
    a = act_ref[b]
    lane = a % 128
    win = inp_ref[0]
    c = win.shape[0]
    lane_ids = lax.broadcasted_iota(jnp.int32, win.shape, 1)
    sel = jnp.sum(jnp.where(lane_ids == lane, win, 0.0), axis=1)
    out_ref[0, 0, :] = sel * (1.0 / jnp.sqrt(jnp.float32(c)))


def _gather_sel(inp_flat, actions):
    b, c, hw = inp_flat.shape
    gs = pltpu.PrefetchScalarGridSpec(
        num_scalar_prefetch=1,
        grid=(b,),
        in_specs=[pl.BlockSpec((1, c, 128),
                               lambda bi, act: (bi, 0, act[bi] // 128))],
        out_specs=pl.BlockSpec((1, 1, c), lambda bi, act: (bi, 0, 0)),
    )
    out = pl.pallas_call(
        _sel_kernel, grid_spec=gs,
        out_shape=jax.ShapeDtypeStruct((b, 1, c), jnp.float32),
    )(actions, inp_flat)
    return out.reshape(b, c)


def _take16(v, idx):
    return jnp.take_along_axis(v, idx, axis=0)


def _cumsum16(x):
    lane = lax.iota(jnp.int32, 16)
    y = x
    for off in (1, 2, 4, 8):
        sh = _take16(y, jnp.maximum(lane - off, 0))
        y = y + jnp.where(lane >= off, sh, jnp.zeros_like(y))
    return y


def _allsum16(x):
    lane = lax.iota(jnp.int32, 16)
    y = x
    for off in (1, 2, 4, 8):
        y = y + _take16(y, lane ^ off)
    return y


def _make_sc_logits(b, hw, c, seg):
    mesh = plsc.VectorSubcoreMesh(core_axis_name="c", subcore_axis_name="s",
                                  num_cores=NC, num_subcores=NS)
    CH = 64
    NCH = seg // CH
    RG = 16

    def body(sel_hbm, enc_hbm, out_hbm, selv, rowbuf, logit_seg, gsem):
        cid = lax.axis_index("c")
        sid = lax.axis_index("s")
        wid = sid * NC + cid
        seg_start = wid * seg
        lane = lax.iota(jnp.int32, 16)

        def batch_body(bi, carry0):
            pltpu.sync_copy(sel_hbm.at[pl.ds(bi * c, c)], selv)
            sch = [selv[pl.ds(16 * kk, 16)] for kk in range(c // 16)]

            pltpu.async_copy(enc_hbm.at[bi, pl.ds(seg_start, CH)],
                             rowbuf.at[0], gsem)

            def chunk_body(ch, carry1):
                slot = ch % 2
                pltpu.make_async_copy(
                    enc_hbm.at[bi, pl.ds(seg_start + ch * CH, CH)],
                    rowbuf.at[slot], gsem).wait()

                @pl.when(ch + 1 < NCH)
                def _next():
                    pltpu.async_copy(
                        enc_hbm.at[bi, pl.ds(seg_start + (ch + 1) * CH, CH)],
                        rowbuf.at[(ch + 1) % 2], gsem)

                def group_body(rg, carry2):
                    vals = jnp.zeros((16,), jnp.float32)
                    for r in range(RG):
                        acc = sch[0] * rowbuf[slot, rg * RG + r,
                                              pl.ds(0, 16)]
                        for kk in range(1, c // 16):
                            acc = acc + sch[kk] * rowbuf[slot, rg * RG + r,
                                                         pl.ds(16 * kk, 16)]
                        vals = jnp.where(lane == r, _allsum16(acc), vals)
                    logit_seg[pl.ds(ch * CH + rg * RG, 16)] = vals
                    return carry2

                lax.fori_loop(0, CH // RG, group_body, 0)
                return carry1

            lax.fori_loop(0, NCH, chunk_body, 0)
            pltpu.sync_copy(logit_seg,
                            out_hbm.at[pl.ds(bi * hw + seg_start, seg)])
            return carry0

        lax.fori_loop(0, b, batch_body, 0)

    kern = pl.kernel(
        body,
        out_type=jax.ShapeDtypeStruct((b * hw,), jnp.float32),
        mesh=mesh,
        compiler_params=pltpu.CompilerParams(use_tc_tiling_on_sc=True, needs_layout_passes=False),
        scratch_types=[
            pltpu.VMEM((c,), jnp.float32),
            pltpu.VMEM((2, CH, c), jnp.float32),
            pltpu.VMEM((seg,), jnp.float32),
            pltpu.SemaphoreType.DMA,
        ],
    )
    return kern


C_CHUNK = 8192
C_ROWS = C_CHUNK // 128


def _loss_kernel(cand_ref, logit_ref, ins_ref, mask_ref, out_ref, acc_ref):
    i = pl.program_id(1)
    nc_ = pl.num_programs(1)

    @pl.when(i == 0)
    def _init():
        acc_ref[...] = jnp.zeros_like(acc_ref)

    pred = jax.nn.sigmoid(logit_ref[0])
    m = (mask_ref[0] > 0.5).astype(jnp.float32)
    g = (ins_ref[0] > 0.5).astype(jnp.float32)

    p = pred * m
    t = g * m
    pt = p * t + (1.0 - p) * (1.0 - t)
    one_m_pt = 1.0 - pt
    focal = -(one_m_pt * one_m_pt) * jnp.log(pt + EPS) * m

    acc_ref[0, :] += jnp.sum(focal, axis=0)
    acc_ref[1, :] += jnp.sum(p * t, axis=0)
    acc_ref[2, :] += jnp.sum(p, axis=0)
    acc_ref[3, :] += jnp.sum(t, axis=0)
    acc_ref[4, :] += jnp.sum(m, axis=0)

    @pl.when(i == nc_ - 1)
    def _fin():
        focal_sum = jnp.sum(acc_ref[0, :])
        inter = jnp.sum(acc_ref[1, :])
        sum_p = jnp.sum(acc_ref[2, :])
        sum_t = jnp.sum(acc_ref[3, :])
        mask_sum = jnp.sum(acc_ref[4, :])
        focal_loss = focal_sum / (mask_sum + EPS)
        dice_loss = 1.0 - (2.0 * inter + EPS) / (sum_p + sum_t + EPS)
        loss_atten = (0.5 * focal_loss + dice_loss) * sum_t
        out_ref[0, 0, :] = jnp.full((128,), loss_atten / (mask_sum + EPS))


def _loss(logits, ins_seg, mask, candidate_idx):
    b, hw = logits.shape
    n_ins = ins_seg.shape[1]
    nc_ = hw // C_CHUNK
    lg = logits.reshape(b, hw // 128, 128)
    ins_rows = ins_seg.reshape(b * n_ins, hw // 128, 128)
    mask_rows = mask.reshape(b, hw // 128, 128)
    gs = pltpu.PrefetchScalarGridSpec(
        num_scalar_prefetch=1,
        grid=(b, nc_),
        in_specs=[
            pl.BlockSpec((1, C_ROWS, 128),
                         lambda bi, ci, cand: (bi, ci, 0)),
            pl.BlockSpec((1, C_ROWS, 128),
                         lambda bi, ci, cand: (bi * n_ins + cand[bi], ci, 0)),
            pl.BlockSpec((1, C_ROWS, 128),
                         lambda bi, ci, cand: (bi, ci, 0)),
        ],
        out_specs=pl.BlockSpec((1, 1, 128), lambda bi, ci, cand: (bi, 0, 0)),
        scratch_shapes=[pltpu.VMEM((8, 128), jnp.float32)],
    )
    out = pl.pallas_call(
        _loss_kernel, grid_spec=gs,
        out_shape=jax.ShapeDtypeStruct((b, 1, 128), jnp.float32),
    )(candidate_idx, lg, ins_rows, mask_rows)
    return out[:, 0, 0]


def kernel(input, encode, ins_seg, mask, actions, candidate_idx):
    b, c, h, w = input.shape
    hw = h * w
    seg = hw // NW

    inp_flat = input.reshape(b, c, hw)
    sel = _gather_sel(inp_flat, actions)
    logits = _make_sc_logits(b, hw, c, seg)(sel.reshape(b * c), encode)
    return _loss(logits.reshape(b, hw), ins_seg, mask, candidate_idx)


# speedup vs baseline: 5.2155x; 5.2155x over previous
import jax
import jax.numpy as jnp
from jax.experimental import pallas as pl
from jax.experimental.pallas import tpu as pltpu

EPS = 1e-6
NSTREAM = 4
SUB = 2048
CHUNK = NSTREAM * SUB
SROWS = SUB // 128


def _kernel(act_ref, cand_ref, inp_ref, *rest):
    enc_refs = rest[:NSTREAM]
    ins_ref, mask_ref, out_ref, acc_ref = rest[NSTREAM:]
    b = pl.program_id(0)
    i = pl.program_id(1)
    nc = pl.num_programs(1)

    @pl.when(i == 0)
    def _init():
        acc_ref[...] = jnp.zeros_like(acc_ref)

    a = act_ref[b]
    si = (a // 128) % 8
    lj = a % 128
    win = inp_ref[0, :, 0]
    c = win.shape[0]
    sub_ids = jax.lax.broadcasted_iota(jnp.int32, win.shape, 1)
    lane_ids = jax.lax.broadcasted_iota(jnp.int32, win.shape, 2)
    hit = (sub_ids == si) & (lane_ids == lj)
    sel = jnp.sum(jnp.where(hit, win, 0.0), axis=(1, 2))

    scale = 1.0 / jnp.sqrt(jnp.float32(c))
    l_parts = []
    for s in range(NSTREAM):
        e = enc_refs[s][0]
        lg = jax.lax.dot_general(
            sel[None, :], e, (((1,), (1,)), ((), ())),
            preferred_element_type=jnp.float32)
        l_parts.append(lg.reshape(SROWS, 128))
    l2 = jnp.concatenate(l_parts, axis=0) * scale
    pred = jax.nn.sigmoid(l2)

    m = (mask_ref[0] > 0.5).astype(jnp.float32)
    g = (ins_ref[0] > 0.5).astype(jnp.float32)

    p = pred * m
    t = g * m
    pt = p * t + (1.0 - p) * (1.0 - t)
    one_m_pt = 1.0 - pt
    focal = -(one_m_pt * one_m_pt) * jnp.log(pt + EPS) * m

    acc_ref[0, :] += jnp.sum(focal, axis=0)
    acc_ref[1, :] += jnp.sum(p * t, axis=0)
    acc_ref[2, :] += jnp.sum(p, axis=0)
    acc_ref[3, :] += jnp.sum(t, axis=0)
    acc_ref[4, :] += jnp.sum(m, axis=0)

    @pl.when(i == nc - 1)
    def _fin():
        focal_sum = jnp.sum(acc_ref[0, :])
        inter = jnp.sum(acc_ref[1, :])
        sum_p = jnp.sum(acc_ref[2, :])
        sum_t = jnp.sum(acc_ref[3, :])
        mask_sum = jnp.sum(acc_ref[4, :])
        focal_loss = focal_sum / (mask_sum + EPS)
        dice_loss = 1.0 - (2.0 * inter + EPS) / (sum_p + sum_t + EPS)
        loss_atten = (0.5 * focal_loss + dice_loss) * sum_t
        out_ref[0, 0, :] = jnp.full((128,), loss_atten / (mask_sum + EPS))


def kernel(input, encode, ins_seg, mask, actions, candidate_idx):
    b, c, h, w = input.shape
    hw = h * w
    n_ins = ins_seg.shape[1]
    nc = hw // CHUNK

    inp4 = input.reshape(b, c, h // 8, 8, w)
    ins_rows = ins_seg.reshape(b * n_ins, hw // 128, 128)
    mask_rows = mask.reshape(b, hw // 128, 128)

    def enc_spec(s):
        return pl.BlockSpec(
            (1, SUB, c),
            lambda bi, ci, act, cand, s=s: (bi, ci * NSTREAM + s, 0))

    grid_spec = pltpu.PrefetchScalarGridSpec(
        num_scalar_prefetch=2,
        grid=(b, nc),
        in_specs=[
            pl.BlockSpec((1, c, 1, 8, 128),
                         lambda bi, ci, act, cand:
                         (bi, 0, act[bi] // 1024, 0, 0)),
            *[enc_spec(s) for s in range(NSTREAM)],
            pl.BlockSpec((1, CHUNK // 128, 128),
                         lambda bi, ci, act, cand:
                         (bi * n_ins + cand[bi], ci, 0)),
            pl.BlockSpec((1, CHUNK // 128, 128),
                         lambda bi, ci, act, cand: (bi, ci, 0)),
        ],
        out_specs=pl.BlockSpec((1, 1, 128),
                               lambda bi, ci, act, cand: (bi, 0, 0)),
        scratch_shapes=[pltpu.VMEM((8, 128), jnp.float32)],
    )

    out = pl.pallas_call(
        _kernel,
        grid_spec=grid_spec,
        out_shape=jax.ShapeDtypeStruct((b, 1, 128), jnp.float32),
    )(actions, candidate_idx, inp4,
      *([encode] * NSTREAM), ins_rows, mask_rows)
    return out[:, 0, 0]


# speedup vs baseline: 5.2224x; 1.0013x over previous
import jax
import jax.numpy as jnp
from jax.experimental import pallas as pl
from jax.experimental.pallas import tpu as pltpu

EPS = 1e-6
NSTREAM = 1
SUB = 8192
CHUNK = NSTREAM * SUB
SROWS = SUB // 128


def _kernel(act_ref, cand_ref, inp_ref, *rest):
    enc_refs = rest[:NSTREAM]
    ins_ref, mask_ref, out_ref, acc_ref = rest[NSTREAM:]
    b = pl.program_id(0)
    i = pl.program_id(1)
    nc = pl.num_programs(1)

    @pl.when(i == 0)
    def _init():
        acc_ref[...] = jnp.zeros_like(acc_ref)

    a = act_ref[b]
    si = (a // 128) % 8
    lj = a % 128
    win = inp_ref[0, :, 0]
    c = win.shape[0]
    sub_ids = jax.lax.broadcasted_iota(jnp.int32, win.shape, 1)
    lane_ids = jax.lax.broadcasted_iota(jnp.int32, win.shape, 2)
    hit = (sub_ids == si) & (lane_ids == lj)
    sel = jnp.sum(jnp.where(hit, win, 0.0), axis=(1, 2))

    scale = 1.0 / jnp.sqrt(jnp.float32(c))
    l_parts = []
    for s in range(NSTREAM):
        e = enc_refs[s][0]
        lg = jax.lax.dot_general(
            sel[None, :], e, (((1,), (1,)), ((), ())),
            preferred_element_type=jnp.float32)
        l_parts.append(lg.reshape(SROWS, 128))
    l2 = jnp.concatenate(l_parts, axis=0) * scale
    pred = jax.nn.sigmoid(l2)

    m = (mask_ref[0] > 0.5).astype(jnp.float32)
    g = (ins_ref[0] > 0.5).astype(jnp.float32)

    p = pred * m
    t = g * m
    pt = p * t + (1.0 - p) * (1.0 - t)
    one_m_pt = 1.0 - pt
    focal = -(one_m_pt * one_m_pt) * jnp.log(pt + EPS) * m

    acc_ref[0, :] += jnp.sum(focal, axis=0)
    acc_ref[1, :] += jnp.sum(p * t, axis=0)
    acc_ref[2, :] += jnp.sum(p, axis=0)
    acc_ref[3, :] += jnp.sum(t, axis=0)
    acc_ref[4, :] += jnp.sum(m, axis=0)

    @pl.when(i == nc - 1)
    def _fin():
        focal_sum = jnp.sum(acc_ref[0, :])
        inter = jnp.sum(acc_ref[1, :])
        sum_p = jnp.sum(acc_ref[2, :])
        sum_t = jnp.sum(acc_ref[3, :])
        mask_sum = jnp.sum(acc_ref[4, :])
        focal_loss = focal_sum / (mask_sum + EPS)
        dice_loss = 1.0 - (2.0 * inter + EPS) / (sum_p + sum_t + EPS)
        loss_atten = (0.5 * focal_loss + dice_loss) * sum_t
        out_ref[0, 0, :] = jnp.full((128,), loss_atten / (mask_sum + EPS))


def kernel(input, encode, ins_seg, mask, actions, candidate_idx):
    b, c, h, w = input.shape
    hw = h * w
    n_ins = ins_seg.shape[1]
    nc = hw // CHUNK

    inp4 = input.reshape(b, c, h // 8, 8, w)
    ins_rows = ins_seg.reshape(b * n_ins, hw // 128, 128)
    mask_rows = mask.reshape(b, hw // 128, 128)

    def enc_spec(s):
        return pl.BlockSpec(
            (1, SUB, c),
            lambda bi, ci, act, cand, s=s: (bi, ci * NSTREAM + s, 0))

    grid_spec = pltpu.PrefetchScalarGridSpec(
        num_scalar_prefetch=2,
        grid=(b, nc),
        in_specs=[
            pl.BlockSpec((1, c, 1, 8, 128),
                         lambda bi, ci, act, cand:
                         (bi, 0, act[bi] // 1024, 0, 0)),
            *[enc_spec(s) for s in range(NSTREAM)],
            pl.BlockSpec((1, CHUNK // 128, 128),
                         lambda bi, ci, act, cand:
                         (bi * n_ins + cand[bi], ci, 0)),
            pl.BlockSpec((1, CHUNK // 128, 128),
                         lambda bi, ci, act, cand: (bi, ci, 0)),
        ],
        out_specs=pl.BlockSpec((1, 1, 128),
                               lambda bi, ci, act, cand: (bi, 0, 0)),
        scratch_shapes=[pltpu.VMEM((8, 128), jnp.float32)],
    )

    out = pl.pallas_call(
        _kernel,
        grid_spec=grid_spec,
        out_shape=jax.ShapeDtypeStruct((b, 1, 128), jnp.float32),
    )(actions, candidate_idx, inp4,
      *([encode] * NSTREAM), ins_rows, mask_rows)
    return out[:, 0, 0]
